# baseline (speedup 1.0000x reference)
import functools

import jax
import jax.numpy as jnp
from jax import lax
from jax.experimental import pallas as pl
from jax.experimental.pallas import tpu as pltpu
from jax.experimental.pallas import tpu_sc as plsc

_NBUF = 4


def _transpose_in_kernel(v, dim, blk):
    n_blocks = (v + blk - 1) // blk
    rows = blk * dim // 128
    half = blk // 2

    def body(i_ref, o_ref):
        xt = i_ref[...].T
        o_ref[:, 0:dim] = xt[0:half]
        o_ref[:, dim : 2 * dim] = xt[half : 2 * half]

    return pl.pallas_call(
        body,
        grid=(n_blocks,),
        in_specs=[pl.BlockSpec((dim, blk), lambda c: (0, c))],
        out_specs=pl.BlockSpec((rows, 128), lambda c: (c, 0)),
        out_shape=jax.ShapeDtypeStruct((n_blocks * rows, 128), jnp.float32),
        compiler_params=pltpu.CompilerParams(vmem_limit_bytes=100 * 1024 * 1024),
    )


def _transpose_out_kernel(b, s, dim, sb, s_half, s_off, alias):
    rows = b * dim // 128
    off_blocks = s_off // sb

    def body(i_ref, *o_refs):
        o_ref = o_refs[-1]
        for j in range(sb):
            xt = i_ref[pl.ds(j * rows, rows), :].T
            o_ref[j] = jnp.concatenate([xt[0:dim], xt[dim : 2 * dim]], axis=1)

    in_specs = [pl.BlockSpec((sb * rows, 128), lambda si: (si, 0))]
    if alias:
        in_specs.append(pl.BlockSpec(memory_space=pl.MemorySpace.ANY))

    return pl.pallas_call(
        body,
        grid=(s_half // sb,),
        in_specs=in_specs,
        out_specs=pl.BlockSpec(
            (sb, dim, b), lambda si: (si + off_blocks, 0, 0)
        ),
        out_shape=jax.ShapeDtypeStruct((s, dim, b), jnp.float32),
        input_output_aliases={1: 0} if alias else {},
        compiler_params=pltpu.CompilerParams(vmem_limit_bytes=100 * 1024 * 1024),
    )


def _gather_kernel(n_half, dim, n_per_w, chunk, n_chunks, nc, n_off):
    mesh = plsc.VectorSubcoreMesh(core_axis_name="c", subcore_axis_name="s")
    n_iters = n_chunks // _NBUF

    @functools.partial(
        pl.kernel,
        mesh=mesh,
        out_type=jax.ShapeDtypeStruct((n_half, dim), jnp.float32),
        scratch_types=[
            pltpu.VMEM((n_per_w,), jnp.int32),
            pltpu.VMEM((_NBUF, chunk, dim), jnp.float32),
            pltpu.SemaphoreType.DMA,
            pltpu.SemaphoreType.DMA,
        ],
        compiler_params=pltpu.CompilerParams(use_tc_tiling_on_sc=False),
    )
    def k(idx_hbm, table_hbm, out_hbm, idx_v, rows_v, gsem, osem):
        wid = lax.axis_index("s") * nc + lax.axis_index("c")
        base = wid * n_per_w
        pltpu.sync_copy(idx_hbm.at[pl.ds(n_off + base, n_per_w)], idx_v)

        def fire_gather(g, b):
            pltpu.async_copy(
                table_hbm.at[idx_v.at[pl.ds(g * chunk, chunk)]],
                rows_v.at[b],
                gsem,
            )

        def fire_out(g, b):
            pltpu.async_copy(
                rows_v.at[b], out_hbm.at[pl.ds(base + g * chunk, chunk)], osem
            )

        def wait_gather(b):
            pltpu.make_async_copy(
                table_hbm.at[idx_v.at[pl.ds(0, chunk)]], rows_v.at[b], gsem
            ).wait()

        def wait_out(b):
            pltpu.make_async_copy(
                rows_v.at[b], out_hbm.at[pl.ds(base, chunk)], osem
            ).wait()

        for b in range(_NBUF - 1):
            fire_gather(b, b)

        def body(i, _):
            for b in range(_NBUF):
                g = i * _NBUF + b
                wait_gather(b)
                if b == 0:
                    @pl.when(i > 0)
                    def _():
                        wait_out(_NBUF - 1)
                    fire_gather(g + _NBUF - 1, _NBUF - 1)
                else:
                    wait_out(b - 1)

                    @pl.when(i < n_iters - 1)
                    def _():
                        fire_gather(g + _NBUF - 1, b - 1)
                fire_out(g, b)
            return 0

        lax.fori_loop(0, n_iters, body, 0)
        wait_out(_NBUF - 1)

    return k


def kernel(input_ids, table):
    b, s = input_ids.shape
    v, dim = table.shape
    n_total = b * s
    nw = 32
    nc = 2
    n_half = n_total // 2
    n_per_w = n_half // nw
    chunk = 400
    n_chunks = n_per_w // chunk
    assert n_per_w * nw == n_half
    assert chunk * n_chunks == n_per_w and n_chunks % _NBUF == 0

    blk = 32768
    t128 = _transpose_in_kernel(v, dim, blk)(table.T)
    v_slots = t128.shape[0] * 128 // dim
    table_rm = t128.reshape(v_slots, dim)

    ids_sm = input_ids.T.astype(jnp.int32)
    ids_perm = ids_sm.reshape(s, 2, b // 2).transpose(0, 2, 1).reshape(n_total)
    half_bits = (blk // 2).bit_length() - 1
    t = ids_perm & (blk - 1)
    slot = (
        (ids_perm & ~(blk - 1)) + ((t & (blk // 2 - 1)) << 1) + (t >> half_bits)
    )

    g1 = _gather_kernel(n_half, dim, n_per_w, chunk, n_chunks, nc, 0)(
        slot, table_rm
    )
    out1 = _transpose_out_kernel(b, s, dim, 4, s // 2, 0, False)(
        g1.reshape(n_half * dim // 128, 128)
    )
    g2 = _gather_kernel(n_half, dim, n_per_w, chunk, n_chunks, nc, n_half)(
        slot, table_rm
    )
    out_t = _transpose_out_kernel(b, s, dim, 4, s // 2, s // 2, True)(
        g2.reshape(n_half * dim // 128, 128), out1
    )
    return out_t.transpose(2, 0, 1)


# speedup vs baseline: 1.4617x; 1.4617x over previous
import functools

import jax
import jax.numpy as jnp
from jax import lax
from jax.experimental import pallas as pl
from jax.experimental.pallas import tpu as pltpu
from jax.experimental.pallas import tpu_sc as plsc

_NBUF = 5


def _transpose_in_kernel(v, dim, blk):
    n_blocks = (v + blk - 1) // blk
    rows = blk * dim // 128
    half = blk // 2

    def body(i_ref, o_ref):
        xt = i_ref[...].T
        o_ref[:, 0:dim] = xt[0:half]
        o_ref[:, dim : 2 * dim] = xt[half : 2 * half]

    return pl.pallas_call(
        body,
        grid=(n_blocks,),
        in_specs=[pl.BlockSpec((dim, blk), lambda c: (0, c))],
        out_specs=pl.BlockSpec((rows, 128), lambda c: (c, 0)),
        out_shape=jax.ShapeDtypeStruct((n_blocks * rows, 128), jnp.float32),
        compiler_params=pltpu.CompilerParams(vmem_limit_bytes=100 * 1024 * 1024),
    )


def _transpose_out_kernel(b, s, dim, sb, s_half, s_off, alias):
    rows = b * dim // 128
    off_blocks = s_off // sb

    def body(i_ref, *o_refs):
        o_ref = o_refs[-1]
        for j in range(sb):
            xt = i_ref[pl.ds(j * rows, rows), :].T
            o_ref[j] = jnp.concatenate([xt[0:dim], xt[dim : 2 * dim]], axis=1)

    in_specs = [pl.BlockSpec((sb * rows, 128), lambda si: (si, 0))]
    if alias:
        in_specs.append(pl.BlockSpec(memory_space=pl.MemorySpace.ANY))

    return pl.pallas_call(
        body,
        grid=(s_half // sb,),
        in_specs=in_specs,
        out_specs=pl.BlockSpec(
            (sb, dim, b), lambda si: (si + off_blocks, 0, 0)
        ),
        out_shape=jax.ShapeDtypeStruct((s, dim, b), jnp.float32),
        input_output_aliases={1: 0} if alias else {},
        compiler_params=pltpu.CompilerParams(vmem_limit_bytes=100 * 1024 * 1024),
    )


def _gather_kernel(n_half, dim, n_per_w, chunk, n_chunks, nc, n_off):
    mesh = plsc.VectorSubcoreMesh(core_axis_name="c", subcore_axis_name="s")
    n_iters = n_chunks // _NBUF

    @functools.partial(
        pl.kernel,
        mesh=mesh,
        out_type=jax.ShapeDtypeStruct((n_half // 2, 2 * dim), jnp.float32),
        scratch_types=[
            pltpu.VMEM((n_per_w,), jnp.int32),
            pltpu.VMEM((_NBUF, chunk, dim), jnp.float32),
            pltpu.SemaphoreType.DMA,
            pltpu.SemaphoreType.DMA,
        ],
        compiler_params=pltpu.CompilerParams(use_tc_tiling_on_sc=False),
    )
    def k(idx_hbm, table_hbm, out_hbm, idx_v, rows_v, gsem, osem):
        wid = lax.axis_index("s") * nc + lax.axis_index("c")
        base = wid * n_per_w
        pltpu.sync_copy(idx_hbm.at[pl.ds(n_off + base, n_per_w)], idx_v)

        def out_slice(g):
            n0 = base + g * chunk
            h = (n0 >> 11) & 1
            row0 = ((n0 >> 12) << 11) + (n0 & 2047)
            return out_hbm.at[pl.ds(row0, chunk), pl.ds(h * dim, dim)]

        def fire_gather(g, b):
            pltpu.async_copy(
                table_hbm.at[idx_v.at[pl.ds(g * chunk, chunk)]],
                rows_v.at[b],
                gsem,
            )

        def fire_out(g, b):
            pltpu.async_copy(rows_v.at[b], out_slice(g), osem)

        def wait_gather(b):
            pltpu.make_async_copy(
                table_hbm.at[idx_v.at[pl.ds(0, chunk)]], rows_v.at[b], gsem
            ).wait()

        def wait_out(b):
            pltpu.make_async_copy(
                rows_v.at[b], out_hbm.at[pl.ds(0, chunk), pl.ds(0, dim)], osem
            ).wait()

        for b in range(_NBUF - 1):
            fire_gather(b, b)

        def body(i, _):
            for b in range(_NBUF):
                g = i * _NBUF + b
                wait_gather(b)
                if b == 0:
                    @pl.when(i > 0)
                    def _():
                        wait_out(_NBUF - 1)
                    fire_gather(g + _NBUF - 1, _NBUF - 1)
                else:
                    wait_out(b - 1)

                    @pl.when(i < n_iters - 1)
                    def _():
                        fire_gather(g + _NBUF - 1, b - 1)
                fire_out(g, b)
            return 0

        lax.fori_loop(0, n_iters, body, 0)
        wait_out(_NBUF - 1)

    return k


def kernel(input_ids, table):
    b, s = input_ids.shape
    v, dim = table.shape
    n_total = b * s
    nw = 32
    nc = 2
    n_half = n_total // 2
    n_per_w = n_half // nw
    chunk = 256
    n_chunks = n_per_w // chunk
    assert n_per_w * nw == n_half
    assert chunk * n_chunks == n_per_w and n_chunks % _NBUF == 0

    blk = 32768
    t128 = _transpose_in_kernel(v, dim, blk)(table.T)
    v_slots = t128.shape[0] * 128 // dim
    table_rm = t128.reshape(v_slots, dim)

    ids_flat = input_ids.T.reshape(n_total).astype(jnp.int32)
    half_bits = (blk // 2).bit_length() - 1
    t = ids_flat & (blk - 1)
    slot = (
        (ids_flat & ~(blk - 1)) + ((t & (blk // 2 - 1)) << 1) + (t >> half_bits)
    )

    g1 = _gather_kernel(n_half, dim, n_per_w, chunk, n_chunks, nc, 0)(
        slot, table_rm
    )
    out1 = _transpose_out_kernel(b, s, dim, 4, s // 2, 0, False)(
        g1.reshape(n_half * dim // 128, 128)
    )
    g2 = _gather_kernel(n_half, dim, n_per_w, chunk, n_chunks, nc, n_half)(
        slot, table_rm
    )
    out_t = _transpose_out_kernel(b, s, dim, 4, s // 2, s // 2, True)(
        g2.reshape(n_half * dim // 128, 128), out1
    )
    return out_t.transpose(2, 0, 1)
